# baseline (speedup 1.0000x reference)
import jax
import jax.numpy as jnp
from jax import lax
from jax.experimental import pallas as pl
from jax.experimental.pallas import tpu as pltpu
from jax.experimental.pallas import tpu_sc as plsc

_NC = 2
_NS = 16
_NW = _NC * _NS
_L = 16

_ROWS = 200
_COLS = 16384
_CW = 128
_NCHUNK = _COLS // (_NW * _CW)
_NBUF = 2


def _sc_body(in_hbm, lut_hbm, out_hbm, table_v, in_v, out_v, in_sems, out_sems):
    wid = lax.axis_index("s") * _NC + lax.axis_index("c")
    base_col = wid * (_NCHUNK * _CW)

    pltpu.sync_copy(lut_hbm, table_v)

    for b in range(_NBUF):
        pltpu.async_copy(
            in_hbm.at[:, pl.ds(base_col + b * _CW, _CW)],
            in_v.at[b],
            in_sems.at[b],
        )

    @pl.loop(0, _NCHUNK, step=_NBUF)
    def _chunk_pair(c0):
        for b in range(_NBUF):
            c_col = base_col + (c0 + b) * _CW
            pltpu.make_async_copy(
                in_hbm.at[:, pl.ds(c_col, _CW)], in_v.at[b], in_sems.at[b]
            ).wait()

            @pl.when(c0 > 0)
            def _drain_prev():
                pltpu.make_async_copy(
                    out_v.at[b],
                    out_hbm.at[:, pl.ds(c_col - _NBUF * _CW, _CW)],
                    out_sems.at[b],
                ).wait()

            @plsc.parallel_loop(0, _ROWS, 1, unroll=4)
            def _gather_rows(r):
                for k in range(_CW // _L):
                    x = in_v[b, r, pl.ds(k * _L, _L)]
                    i = (x * 255.0).astype(jnp.int32)
                    t = ((i << 1) ^ ((i >> 7) * 27)) & 255
                    out_v[b, r, pl.ds(k * _L, _L)] = t.astype(jnp.float32) / 255.0

            pltpu.async_copy(
                out_v.at[b], out_hbm.at[:, pl.ds(c_col, _CW)], out_sems.at[b]
            )

            @pl.when(c0 + _NBUF < _NCHUNK)
            def _prefetch_next():
                pltpu.async_copy(
                    in_hbm.at[:, pl.ds(c_col + _NBUF * _CW, _CW)],
                    in_v.at[b],
                    in_sems.at[b],
                )

    for c in range(_NCHUNK - _NBUF, _NCHUNK):
        b = c % _NBUF
        pltpu.make_async_copy(
            out_v.at[b],
            out_hbm.at[:, pl.ds(base_col + c * _CW, _CW)],
            out_sems.at[b],
        ).wait()


@jax.jit
def _run(inputs_t, lookup):
    mesh = plsc.VectorSubcoreMesh(core_axis_name="c", subcore_axis_name="s")
    return pl.kernel(
        _sc_body,
        out_type=jax.ShapeDtypeStruct((_ROWS, _COLS), jnp.float32),
        mesh=mesh,
        scratch_types=[
            pltpu.VMEM((256,), jnp.float32),
            pltpu.VMEM((_NBUF, _ROWS, _CW), jnp.float32),
            pltpu.VMEM((_NBUF, _ROWS, _CW), jnp.float32),
            pltpu.SemaphoreType.DMA((_NBUF,)),
            pltpu.SemaphoreType.DMA((_NBUF,)),
        ],
        compiler_params=pltpu.CompilerParams(
            needs_layout_passes=False, use_tc_tiling_on_sc=True
        ),
    )(inputs_t, lookup)


def kernel(inputs, lookup):
    return _run(inputs.T, lookup).T


# speedup vs baseline: 1.1708x; 1.1708x over previous
import jax
import jax.numpy as jnp
from jax import lax
from jax.experimental import pallas as pl
from jax.experimental.pallas import tpu as pltpu
from jax.experimental.pallas import tpu_sc as plsc

_NC = 2
_NS = 16
_NW = _NC * _NS
_L = 16

_ROWS = 200
_COLS = 16384
_CW = 128
_NCHUNK = _COLS // (_NW * _CW)
_NBUF = 2


def _sc_body(in_hbm, lut_hbm, out_hbm, table_v, in_v, out_v, in_sems, out_sems):
    wid = lax.axis_index("s") * _NC + lax.axis_index("c")
    base_col = wid * (_NCHUNK * _CW)

    for b in range(_NBUF):
        pltpu.async_copy(
            in_hbm.at[:, pl.ds(base_col + b * _CW, _CW)],
            in_v.at[b],
            in_sems.at[b],
        )
    pltpu.sync_copy(lut_hbm, table_v)

    @pl.loop(0, _NCHUNK, step=_NBUF)
    def _chunk_pair(c0):
        for b in range(_NBUF):
            c_col = base_col + (c0 + b) * _CW
            pltpu.make_async_copy(
                in_hbm.at[:, pl.ds(c_col, _CW)], in_v.at[b], in_sems.at[b]
            ).wait()

            @pl.when(c0 > 0)
            def _drain_prev():
                pltpu.make_async_copy(
                    out_v.at[b],
                    out_hbm.at[:, pl.ds(c_col - _NBUF * _CW, _CW)],
                    out_sems.at[b],
                ).wait()

            @plsc.parallel_loop(0, _ROWS, 1, unroll=4)
            def _gather_rows(r):
                for k in range(_CW // _L):
                    x = in_v[b, r, pl.ds(k * _L, _L)]
                    idx = plsc.bitcast((x * 255.0).astype(jnp.int32), jnp.uint32)
                    idx = plsc.bitcast(jnp.minimum(idx, 255), jnp.int32)
                    out_v[b, r, pl.ds(k * _L, _L)] = plsc.load_gather(
                        table_v, [idx]
                    )

            pltpu.async_copy(
                out_v.at[b], out_hbm.at[:, pl.ds(c_col, _CW)], out_sems.at[b]
            )

            @pl.when(c0 + _NBUF < _NCHUNK)
            def _prefetch_next():
                pltpu.async_copy(
                    in_hbm.at[:, pl.ds(c_col + _NBUF * _CW, _CW)],
                    in_v.at[b],
                    in_sems.at[b],
                )

    for c in range(_NCHUNK - _NBUF, _NCHUNK):
        b = c % _NBUF
        pltpu.make_async_copy(
            out_v.at[b],
            out_hbm.at[:, pl.ds(base_col + c * _CW, _CW)],
            out_sems.at[b],
        ).wait()


@jax.jit
def _run(inputs_t, lookup):
    mesh = plsc.VectorSubcoreMesh(core_axis_name="c", subcore_axis_name="s")
    return pl.kernel(
        _sc_body,
        out_type=jax.ShapeDtypeStruct((_ROWS, _COLS), jnp.float32),
        mesh=mesh,
        scratch_types=[
            pltpu.VMEM((256,), jnp.float32),
            pltpu.VMEM((_NBUF, _ROWS, _CW), jnp.float32),
            pltpu.VMEM((_NBUF, _ROWS, _CW), jnp.float32),
            pltpu.SemaphoreType.DMA((_NBUF,)),
            pltpu.SemaphoreType.DMA((_NBUF,)),
        ],
        compiler_params=pltpu.CompilerParams(
            needs_layout_passes=False, use_tc_tiling_on_sc=True
        ),
    )(inputs_t, lookup)


def kernel(inputs, lookup):
    return _run(inputs.T, lookup).T
